# baseline (device time: 737274 ns/iter reference)
import jax
import jax.numpy as jnp
from jax import lax
from jax.experimental import pallas as pl
from jax.experimental.pallas import tpu as pltpu

N_DEV = 4
M, K, N = 4096, 4096, 8192
KS = K // N_DEV
BLK = M // N_DEV
CH = 512
CPB = BLK // CH
HALF = N // 2
STG = 128
N_HOP = 2 * (N_DEV - 1)


def _ar_body(x_ref, w_ref, out_ref,
             comm_r, part_r, seed_r, comm_l, part_l, seed_l, stage,
             send_r, recv_r, send_l, recv_l, store_sem):
    my = lax.axis_index("i")
    left = lax.rem(my + N_DEV - 1, N_DEV)
    right = lax.rem(my + 1, N_DEV)

    barrier = pltpu.get_barrier_semaphore()
    for nbr in (left, right):
        pl.semaphore_signal(barrier, inc=1, device_id=(nbr,),
                            device_id_type=pl.DeviceIdType.MESH)
    pl.semaphore_wait(barrier, 2)

    rings = (
        dict(idx=0, nbr=right, sgn=-1, comm=comm_r, part=part_r,
             seed=seed_r, send=send_r, recv=recv_r, col0=0),
        dict(idx=1, nbr=left, sgn=+1, comm=comm_l, part=part_l,
             seed=seed_l, send=send_l, recv=recv_l, col0=HALF),
    )

    def block(b):
        return lax.rem(b + 2 * N_DEV, N_DEV)

    def gemm(blk, c, r):
        return jnp.dot(
            x_ref[pl.ds(blk * BLK + c * CH, CH), :],
            w_ref[:, pl.ds(r["col0"], HALF)],
            preferred_element_type=jnp.float32,
        ).astype(jnp.bfloat16)

    in_flight_store = []

    def do_store(r, r_slot, store_blk, c):
        for p in range(CH // STG):
            while in_flight_store:
                in_flight_store.pop().wait()
            stage[...] = r["comm"][
                r_slot, pl.ds(p * STG, STG), :].astype(jnp.float32)
            st = pltpu.make_async_copy(
                stage,
                out_ref.at[pl.ds(store_blk * BLK + c * CH + p * STG, STG),
                           pl.ds(r["col0"], HALF)],
                store_sem)
            st.start()
            in_flight_store.append(st)

    for c in range(CPB):
        deferred = []
        for h in range(N_HOP):
            s_slot, r_slot = h % 2, (h + 1) % 2
            rdmas = []
            for r in rings:
                if h == 0 and c == 0:
                    r["seed"][...] = gemm(my, c, r)
                rdma = pltpu.make_async_remote_copy(
                    src_ref=r["seed"] if h == 0 else r["comm"].at[s_slot],
                    dst_ref=r["comm"].at[r_slot],
                    send_sem=r["send"].at[s_slot],
                    recv_sem=r["recv"].at[r_slot],
                    device_id=(r["nbr"],),
                    device_id_type=pl.DeviceIdType.MESH)
                rdma.start()
                rdmas.append(rdma)
            if h < N_DEV - 1:
                for r in rings:
                    r["part"][...] = gemm(block(my + r["sgn"] * (h + 1)), c, r)
            elif h == N_DEV - 1 and c + 1 < CPB:
                for r in rings:
                    r["seed"][...] = gemm(my, c + 1, r)
            for args in deferred:
                do_store(*args, c)
            deferred = []
            for rdma in rdmas:
                rdma.wait()
            for r in rings:
                if h < N_DEV - 1:
                    r["comm"][r_slot] = (
                        r["comm"][r_slot].astype(jnp.float32)
                        + r["part"][...].astype(jnp.float32)
                    ).astype(jnp.bfloat16)
                    if h == N_DEV - 2:
                        deferred.append(
                            (r, r_slot, block(my + r["sgn"] * (h + 1))))
                else:
                    deferred.append(
                        (r, r_slot, block(my + r["sgn"] * (h - (N_DEV - 1)))))
        for args in deferred:
            do_store(*args, c)

    while in_flight_store:
        in_flight_store.pop().wait()


def kernel(x, w_mat, scale_x, scale_w):
    s = (scale_x[0] * scale_w[0]).astype(jnp.float32)
    xs = (x.astype(jnp.float32) * s).astype(jnp.bfloat16)
    ws = w_mat.astype(jnp.bfloat16)

    return pl.pallas_call(
        _ar_body,
        out_shape=jax.ShapeDtypeStruct((M, N), jnp.float32),
        in_specs=[
            pl.BlockSpec(memory_space=pltpu.MemorySpace.VMEM),
            pl.BlockSpec(memory_space=pltpu.MemorySpace.VMEM),
        ],
        out_specs=pl.BlockSpec(memory_space=pltpu.MemorySpace.HBM),
        scratch_shapes=[
            pltpu.VMEM((2, CH, HALF), jnp.bfloat16),
            pltpu.VMEM((CH, HALF), jnp.bfloat16),
            pltpu.VMEM((CH, HALF), jnp.bfloat16),
            pltpu.VMEM((2, CH, HALF), jnp.bfloat16),
            pltpu.VMEM((CH, HALF), jnp.bfloat16),
            pltpu.VMEM((CH, HALF), jnp.bfloat16),
            pltpu.VMEM((STG, HALF), jnp.float32),
            pltpu.SemaphoreType.DMA((2,)),
            pltpu.SemaphoreType.DMA((2,)),
            pltpu.SemaphoreType.DMA((2,)),
            pltpu.SemaphoreType.DMA((2,)),
            pltpu.SemaphoreType.DMA,
        ],
        compiler_params=pltpu.CompilerParams(
            collective_id=0, vmem_limit_bytes=100 * 1024 * 1024),
    )(xs, ws)


# device time: 685854 ns/iter; 1.0750x vs baseline; 1.0750x over previous
import jax
import jax.numpy as jnp
from jax import lax
from jax.experimental import pallas as pl
from jax.experimental.pallas import tpu as pltpu

N_DEV = 4
M, K, N = 4096, 4096, 8192
KS = K // N_DEV
BLK = M // N_DEV
CH = 512
CPB = BLK // CH
HALF = N // 2
SUB = 4
SW = CH // SUB
STG = 128
N_HOP = 2 * (N_DEV - 1)
RS_HOPS = N_DEV - 1


def _ar_body(x_ref, w_ref, out_ref,
             comm_r, part_r, seed_r, comm_l, part_l, seed_l, stage,
             send_r, recv_r, send_l, recv_l, store_sem):
    my = lax.axis_index("i")
    left = lax.rem(my + N_DEV - 1, N_DEV)
    right = lax.rem(my + 1, N_DEV)

    barrier = pltpu.get_barrier_semaphore()
    for nbr in (left, right):
        pl.semaphore_signal(barrier, inc=1, device_id=(nbr,),
                            device_id_type=pl.DeviceIdType.MESH)
    pl.semaphore_wait(barrier, 2)

    rings = (
        dict(idx=0, nbr=right, sgn=-1, comm=comm_r, part=part_r,
             seed=seed_r, send=send_r, recv=recv_r, col0=0),
        dict(idx=1, nbr=left, sgn=+1, comm=comm_l, part=part_l,
             seed=seed_l, send=send_l, recv=recv_l, col0=HALF),
    )

    def block(b):
        return lax.rem(b + 2 * N_DEV, N_DEV)

    def gemm(blk, c, r):
        return jnp.dot(
            x_ref[pl.ds(blk * BLK + c * CH, CH), :],
            w_ref[:, pl.ds(r["col0"], HALF)],
            preferred_element_type=jnp.float32,
        ).astype(jnp.bfloat16)

    in_flight_store = []

    def do_store(r, r_slot, store_blk, c):
        for p in range(CH // STG):
            while in_flight_store:
                in_flight_store.pop().wait()
            stage[...] = r["comm"][
                r_slot, pl.ds(p * STG, STG), :].astype(jnp.float32)
            st = pltpu.make_async_copy(
                stage,
                out_ref.at[pl.ds(store_blk * BLK + c * CH + p * STG, STG),
                           pl.ds(r["col0"], HALF)],
                store_sem)
            st.start()
            in_flight_store.append(st)

    last_send = {}
    cur_recv = {}

    def issue(c, h, s, r):
        sl, rl = h % 2, (h + 1) % 2
        if h == 0:
            src = r["seed"].at[pl.ds(s * SW, SW)]
        else:
            src = r["comm"].at[sl, pl.ds(s * SW, SW)]
        key = (r["idx"], sl, s)
        if key in last_send:
            last_send[key].wait_send()
        rd = pltpu.make_async_remote_copy(
            src_ref=src,
            dst_ref=r["comm"].at[rl, pl.ds(s * SW, SW)],
            send_sem=r["send"].at[sl, s],
            recv_sem=r["recv"].at[rl, s],
            device_id=(r["nbr"],),
            device_id_type=pl.DeviceIdType.MESH)
        rd.start()
        last_send[key] = rd
        cur_next[(r["idx"], s)] = rd

    seq = [(c, h) for c in range(CPB) for h in range(N_HOP)]

    for r in rings:
        r["seed"][...] = gemm(my, 0, r)
    cur_next = {}
    for s in range(SUB):
        for r in rings:
            issue(0, 0, s, r)

    deferred = []
    for idx, (c, h) in enumerate(seq):
        sl, rl = h % 2, (h + 1) % 2
        cur_recv, cur_next = cur_next, {}
        if h < RS_HOPS:
            for r in rings:
                r["part"][...] = gemm(block(my + r["sgn"] * (h + 1)), c, r)
        elif h == RS_HOPS and c + 1 < CPB:
            for r in rings:
                r["seed"][...] = gemm(my, c + 1, r)
        for args in deferred:
            do_store(*args)
        deferred = []

        nxt = seq[idx + 1] if idx + 1 < len(seq) else None
        for s in range(SUB):
            for r in rings:
                cur_recv[(r["idx"], s)].wait_recv()
            if h < RS_HOPS:
                for r in rings:
                    r["comm"][rl, pl.ds(s * SW, SW), :] = (
                        r["comm"][rl, pl.ds(s * SW, SW), :]
                        .astype(jnp.float32)
                        + r["part"][pl.ds(s * SW, SW), :].astype(jnp.float32)
                    ).astype(jnp.bfloat16)
            if nxt is not None:
                for r in rings:
                    issue(nxt[0], nxt[1], s, r)
        for r in rings:
            if h == RS_HOPS - 1:
                deferred.append(
                    (r, rl, block(my + r["sgn"] * (h + 1)), c))
            elif h >= RS_HOPS:
                deferred.append(
                    (r, rl, block(my + r["sgn"] * (h - RS_HOPS)), c))
    for args in deferred:
        do_store(*args)

    for rd in last_send.values():
        rd.wait_send()

    while in_flight_store:
        in_flight_store.pop().wait()


def kernel(x, w_mat, scale_x, scale_w):
    s = (scale_x[0] * scale_w[0]).astype(jnp.float32)
    xs = (x.astype(jnp.float32) * s).astype(jnp.bfloat16)
    ws = w_mat.astype(jnp.bfloat16)

    return pl.pallas_call(
        _ar_body,
        out_shape=jax.ShapeDtypeStruct((M, N), jnp.float32),
        in_specs=[
            pl.BlockSpec(memory_space=pltpu.MemorySpace.VMEM),
            pl.BlockSpec(memory_space=pltpu.MemorySpace.VMEM),
        ],
        out_specs=pl.BlockSpec(memory_space=pltpu.MemorySpace.HBM),
        scratch_shapes=[
            pltpu.VMEM((2, CH, HALF), jnp.bfloat16),
            pltpu.VMEM((CH, HALF), jnp.bfloat16),
            pltpu.VMEM((CH, HALF), jnp.bfloat16),
            pltpu.VMEM((2, CH, HALF), jnp.bfloat16),
            pltpu.VMEM((CH, HALF), jnp.bfloat16),
            pltpu.VMEM((CH, HALF), jnp.bfloat16),
            pltpu.VMEM((STG, HALF), jnp.float32),
            pltpu.SemaphoreType.DMA((2, SUB)),
            pltpu.SemaphoreType.DMA((2, SUB)),
            pltpu.SemaphoreType.DMA((2, SUB)),
            pltpu.SemaphoreType.DMA((2, SUB)),
            pltpu.SemaphoreType.DMA,
        ],
        compiler_params=pltpu.CompilerParams(
            collective_id=0, vmem_limit_bytes=100 * 1024 * 1024),
    )(xs, ws)
